# baseline (speedup 1.0000x reference)
import functools

import jax
import jax.numpy as jnp
from jax import lax
from jax.experimental import pallas as pl
from jax.experimental.pallas import tpu as pltpu
from jax.experimental.pallas import tpu_sc as plsc

_CHUNK = 128


def _sc_gather3(user3, item13, item23, U_MF, I_MF_1, I_MF_2, B, D, NC, NS):
    NW = NC * NS
    bpw = B // NW
    n_chunks = bpw // _CHUNK
    mesh = plsc.VectorSubcoreMesh(core_axis_name="c", subcore_axis_name="s")

    @functools.partial(
        pl.kernel,
        out_type=[jax.ShapeDtypeStruct((B, D), jnp.float32)] * 3,
        mesh=mesh,
        scratch_types=[pltpu.VMEM((n_chunks, _CHUNK), jnp.int32)] * 3
        + [pltpu.VMEM((bpw, D), jnp.float32)] * 3
        + [pltpu.SemaphoreType.DMA],
        compiler_params=pltpu.CompilerParams(use_tc_tiling_on_sc=False),
    )
    def k(u_hbm, i1_hbm, i2_hbm, tu_hbm, t1_hbm, t2_hbm,
          uo, o1, o2, uix, ix1, ix2, ur, r1, r2, sem):
        wid = lax.axis_index("s") * NC + lax.axis_index("c")
        base = wid * bpw
        pltpu.sync_copy(u_hbm.at[wid], uix)
        pltpu.sync_copy(i1_hbm.at[wid], ix1)
        pltpu.sync_copy(i2_hbm.at[wid], ix2)
        copies = []
        for tbl, ix, dst in ((tu_hbm, uix, ur), (t1_hbm, ix1, r1), (t2_hbm, ix2, r2)):
            for j in range(n_chunks):
                copies.append(
                    pltpu.async_copy(
                        tbl.at[ix.at[j]], dst.at[pl.ds(j * _CHUNK, _CHUNK)], sem
                    )
                )
        for c in copies:
            c.wait()
        pltpu.sync_copy(ur, uo.at[pl.ds(base, bpw)])
        pltpu.sync_copy(r1, o1.at[pl.ds(base, bpw)])
        pltpu.sync_copy(r2, o2.at[pl.ds(base, bpw)])

    return k(user3, item13, item23, U_MF, I_MF_1, I_MF_2)


def _tc_mlp_body(u_ref, i1_ref, i2_ref, w10, b10, w11, b11, w20, b20, w21, b21,
                 o1_ref, o2_ref):
    u = u_ref[0]
    e1 = u * i1_ref[0]
    e2 = u * i2_ref[0]
    h1 = jnp.maximum(
        jnp.dot(e1, w10[...], preferred_element_type=jnp.float32) + b10[...], 0.0)
    h2 = jnp.maximum(
        jnp.dot(e2, w20[...], preferred_element_type=jnp.float32) + b20[...], 0.0)
    s1 = jnp.sum(h1 * w11[...], axis=1, keepdims=True) + b11[...]
    s2 = jnp.sum(h2 * w21[...], axis=1, keepdims=True) + b21[...]
    o1_ref[0, 0] = jnp.maximum(s1, 0.0)[:, 0]
    o2_ref[0, 0] = jnp.maximum(s2, 0.0)[:, 0]


def _tc_mlp(u_e, i1_e, i2_e, W1_0, b1_0, W1_1, b1_1, W2_0, b2_0, W2_1, b2_1):
    B, D = u_e.shape
    NB = 8
    BLK = B // NB
    H = W1_0.shape[1]
    row = lambda i: (i, 0, 0)
    fixed2 = lambda i: (0, 0)
    in_specs = [
        pl.BlockSpec((1, BLK, D), row),
        pl.BlockSpec((1, BLK, D), row),
        pl.BlockSpec((1, BLK, D), row),
        pl.BlockSpec((D, H), fixed2),
        pl.BlockSpec((1, H), fixed2),
        pl.BlockSpec((1, H), fixed2),
        pl.BlockSpec((1, 1), fixed2),
        pl.BlockSpec((D, H), fixed2),
        pl.BlockSpec((1, H), fixed2),
        pl.BlockSpec((1, H), fixed2),
        pl.BlockSpec((1, 1), fixed2),
    ]
    out_specs = [
        pl.BlockSpec((1, 1, BLK), row),
        pl.BlockSpec((1, 1, BLK), row),
    ]
    o1, o2 = pl.pallas_call(
        _tc_mlp_body,
        grid=(NB,),
        in_specs=in_specs,
        out_specs=out_specs,
        out_shape=[jax.ShapeDtypeStruct((NB, 1, BLK), jnp.float32)] * 2,
    )(
        u_e.reshape(NB, BLK, D),
        i1_e.reshape(NB, BLK, D),
        i2_e.reshape(NB, BLK, D),
        W1_0, b1_0.reshape(1, H), W1_1.reshape(1, H), b1_1.reshape(1, 1),
        W2_0, b2_0.reshape(1, H), W2_1.reshape(1, H), b2_1.reshape(1, 1),
    )
    return o1.reshape(B), o2.reshape(B)


def kernel(user, item1, item2, U_MF, I_MF_1, I_MF_2,
           W1_0, b1_0, W1_1, b1_1, W2_0, b2_0, W2_1, b2_1):
    B = user.shape[0]
    D = U_MF.shape[1]
    info = plsc.get_sparse_core_info()
    NC, NS = info.num_cores, info.num_subcores
    NW = NC * NS
    n_chunks = (B // NW) // _CHUNK
    shp = (NW, n_chunks, _CHUNK)
    u3 = user.astype(jnp.int32).reshape(shp)
    it13 = item1.astype(jnp.int32).reshape(shp)
    it23 = item2.astype(jnp.int32).reshape(shp)
    u_e, i1_e, i2_e = _sc_gather3(u3, it13, it23, U_MF, I_MF_1, I_MF_2,
                                  B, D, NC, NS)
    o1, o2 = _tc_mlp(u_e, i1_e, i2_e,
                     W1_0, b1_0, W1_1, b1_1, W2_0, b2_0, W2_1, b2_1)
    return o1, o2, u_e, i1_e, i2_e
